# baseline (speedup 1.0000x reference)
import jax
import jax.numpy as jnp
from jax import lax
from jax.experimental import pallas as pl
from jax.experimental.pallas import tpu as pltpu
from jax.experimental.pallas import tpu_sc as plsc

N = 10000
E = 320000
EP = 327680
D = 128
NP = 10240
NC = 2
NS = 16
NW = NC * NS
EPW = EP // NW
CH = 128
NCHUNK = EPW // CH
RPT = NP // NS
BR = 640

_f32 = jnp.float32
_mesh = plsc.VectorSubcoreMesh(
    core_axis_name="c", subcore_axis_name="s", num_cores=NC, num_subcores=NS
)


def _deg_body(src_hbm, dst_hbm, od_out, id_out,
              si0, si1, di0, di1, ones_v, z16_v, cb_v,
              ssi0, ssi1, sdi0, sdi1, sp_od, sp_id):
    c = lax.axis_index("c")
    s = lax.axis_index("s")
    wid = s * NC + c
    base = wid * EPW

    z16_v[...] = jnp.zeros((16,), _f32)
    for i in range(CH // 16):
        ones_v[pl.ds(16 * i, 16)] = jnp.ones((16,), _f32)

    def _z(k, _):
        pltpu.sync_copy(z16_v, sp_od.at[pl.ds(s * RPT + 16 * k, 16)])
        pltpu.sync_copy(z16_v, sp_id.at[pl.ds(s * RPT + 16 * k, 16)])
        return _
    lax.fori_loop(0, RPT // 16, _z, None)
    plsc.subcore_barrier()

    def _idx(j, sv, dv, ss, sd):
        pltpu.async_copy(src_hbm.at[pl.ds(base + j * CH, CH)], sv, ss)
        pltpu.async_copy(dst_hbm.at[pl.ds(base + j * CH, CH)], dv, sd)

    def _wi(v, sem):
        pltpu.make_async_copy(src_hbm.at[pl.ds(0, CH)], v, sem).wait()

    _idx(0, si0, di0, ssi0, sdi0)
    _idx(1, si1, di1, ssi1, sdi1)

    def _pipe(t, _):
        j0 = 2 * t
        _wi(si0, ssi0)
        pltpu.sync_copy(ones_v, sp_od.at[si0], add=True)
        _wi(di0, sdi0)
        pltpu.sync_copy(ones_v, sp_id.at[di0], add=True)
        _idx(j0 + 2, si0, di0, ssi0, sdi0)
        _wi(si1, ssi1)
        pltpu.sync_copy(ones_v, sp_od.at[si1], add=True)
        _wi(di1, sdi1)
        pltpu.sync_copy(ones_v, sp_id.at[di1], add=True)
        _idx(j0 + 3, si1, di1, ssi1, sdi1)
        return _
    lax.fori_loop(0, NCHUNK // 2 - 1, _pipe, None)

    _wi(si0, ssi0)
    pltpu.sync_copy(ones_v, sp_od.at[si0], add=True)
    _wi(di0, sdi0)
    pltpu.sync_copy(ones_v, sp_id.at[di0], add=True)
    _wi(si1, ssi1)
    pltpu.sync_copy(ones_v, sp_od.at[si1], add=True)
    _wi(di1, sdi1)
    pltpu.sync_copy(ones_v, sp_id.at[di1], add=True)
    plsc.subcore_barrier()

    pltpu.sync_copy(sp_od.at[pl.ds(s * RPT, RPT)], cb_v)
    pltpu.sync_copy(cb_v, od_out.at[c, pl.ds(s * RPT, RPT)])
    pltpu.sync_copy(sp_id.at[pl.ds(s * RPT, RPT)], cb_v)
    pltpu.sync_copy(cb_v, id_out.at[c, pl.ds(s * RPT, RPT)])


_sc_deg = pl.kernel(
    _deg_body,
    out_type=[jax.ShapeDtypeStruct((NC, NP), _f32),
              jax.ShapeDtypeStruct((NC, NP), _f32)],
    mesh=_mesh,
    scratch_types=[
        pltpu.VMEM((CH,), jnp.int32),
        pltpu.VMEM((CH,), jnp.int32),
        pltpu.VMEM((CH,), jnp.int32),
        pltpu.VMEM((CH,), jnp.int32),
        pltpu.VMEM((CH,), _f32),
        pltpu.VMEM((16,), _f32),
        pltpu.VMEM((RPT,), _f32),
        pltpu.SemaphoreType.DMA,
        pltpu.SemaphoreType.DMA,
        pltpu.SemaphoreType.DMA,
        pltpu.SemaphoreType.DMA,
        pltpu.VMEM_SHARED((NP,), _f32),
        pltpu.VMEM_SHARED((NP,), _f32),
    ],
)


def _agg_body(h_hbm, src_hbm, dst_hbm, part_out,
              si0, si1, di0, di1, rb0, rb1, zb_v,
              ssi0, ssi1, sdi0, sdi1, sg0, sg1, sp_agg):
    c = lax.axis_index("c")
    s = lax.axis_index("s")
    wid = s * NC + c
    base = wid * EPW

    for r in range(16):
        for q in range(D // 16):
            zb_v[r, pl.ds(16 * q, 16)] = jnp.zeros((16,), _f32)

    def _z(k, _):
        pltpu.sync_copy(zb_v, sp_agg.at[pl.ds(s * RPT + 16 * k, 16)])
        return _
    lax.fori_loop(0, RPT // 16, _z, None)
    plsc.subcore_barrier()

    def _idx(j, sv, dv, ss, sd):
        pltpu.async_copy(src_hbm.at[pl.ds(base + j * CH, CH)], sv, ss)
        pltpu.async_copy(dst_hbm.at[pl.ds(base + j * CH, CH)], dv, sd)

    def _wi(v, sem):
        pltpu.make_async_copy(src_hbm.at[pl.ds(0, CH)], v, sem).wait()

    def _wg(rb, sem):
        pltpu.make_async_copy(h_hbm.at[pl.ds(0, CH)], rb, sem).wait()

    _idx(0, si0, di0, ssi0, sdi0)
    _wi(si0, ssi0)
    pltpu.async_copy(h_hbm.at[si0], rb0, sg0)
    _idx(1, si1, di1, ssi1, sdi1)

    def _pipe(t, _):
        j0 = 2 * t
        _wi(si1, ssi1)
        pltpu.async_copy(h_hbm.at[si1], rb1, sg1)
        _wg(rb0, sg0)
        _wi(di0, sdi0)
        pltpu.sync_copy(rb0, sp_agg.at[di0], add=True)
        _idx(j0 + 2, si0, di0, ssi0, sdi0)
        _wg(rb1, sg1)
        _wi(di1, sdi1)
        pltpu.sync_copy(rb1, sp_agg.at[di1], add=True)
        _wi(si0, ssi0)
        pltpu.async_copy(h_hbm.at[si0], rb0, sg0)
        _idx(j0 + 3, si1, di1, ssi1, sdi1)
        return _
    lax.fori_loop(0, NCHUNK // 2 - 1, _pipe, None)

    _wi(si1, ssi1)
    pltpu.async_copy(h_hbm.at[si1], rb1, sg1)
    _wg(rb0, sg0)
    _wi(di0, sdi0)
    pltpu.sync_copy(rb0, sp_agg.at[di0], add=True)
    _wg(rb1, sg1)
    _wi(di1, sdi1)
    pltpu.sync_copy(rb1, sp_agg.at[di1], add=True)
    plsc.subcore_barrier()

    def _out(k, _):
        b2 = s * RPT + CH * k
        pltpu.sync_copy(sp_agg.at[pl.ds(b2, CH)], rb0)
        pltpu.sync_copy(rb0, part_out.at[c, pl.ds(b2, CH)])
        return _
    lax.fori_loop(0, RPT // CH, _out, None)


_sc_agg = pl.kernel(
    _agg_body,
    out_type=jax.ShapeDtypeStruct((NC, NP, D), _f32),
    mesh=_mesh,
    scratch_types=[
        pltpu.VMEM((CH,), jnp.int32),
        pltpu.VMEM((CH,), jnp.int32),
        pltpu.VMEM((CH,), jnp.int32),
        pltpu.VMEM((CH,), jnp.int32),
        pltpu.VMEM((CH, D), _f32),
        pltpu.VMEM((CH, D), _f32),
        pltpu.VMEM((16, D), _f32),
        pltpu.SemaphoreType.DMA,
        pltpu.SemaphoreType.DMA,
        pltpu.SemaphoreType.DMA,
        pltpu.SemaphoreType.DMA,
        pltpu.SemaphoreType.DMA,
        pltpu.SemaphoreType.DMA,
        pltpu.VMEM_SHARED((NP, D), _f32),
    ],
)


def _scale_body(x_ref, odp_ref, idp_ref, hs_ref, ns_ref, nd_ref):
    od = odp_ref[0] + odp_ref[1]
    ind = idp_ref[0] + idp_ref[1]
    ns = lax.rsqrt(jnp.maximum(od, 1.0))
    nd = lax.rsqrt(jnp.maximum(ind, 1.0))
    i = pl.program_id(0)
    rows = i * BR + lax.broadcasted_iota(jnp.int32, (BR, 1), 0)
    hs_ref[...] = jnp.where(rows < N, x_ref[...] * ns, 0.0)
    ns_ref[...] = ns
    nd_ref[...] = nd


def _tc_scale(x, odp, idp):
    return pl.pallas_call(
        _scale_body,
        grid=(NP // BR,),
        in_specs=[
            pl.BlockSpec((BR, D), lambda i: (i, 0)),
            pl.BlockSpec((NC, BR, 1), lambda i: (0, i, 0)),
            pl.BlockSpec((NC, BR, 1), lambda i: (0, i, 0)),
        ],
        out_specs=[
            pl.BlockSpec((BR, D), lambda i: (i, 0)),
            pl.BlockSpec((BR, 1), lambda i: (i, 0)),
            pl.BlockSpec((BR, 1), lambda i: (i, 0)),
        ],
        out_shape=[
            jax.ShapeDtypeStruct((NP, D), _f32),
            jax.ShapeDtypeStruct((NP, 1), _f32),
            jax.ShapeDtypeStruct((NP, 1), _f32),
        ],
    )(x, odp, idp)


def _layer_body(part_ref, nd_ref, w_ref, b_ref, ns_ref, out_ref, hs_ref):
    agg = (part_ref[0] + part_ref[1]) * nd_ref[...]
    o = jnp.dot(agg, w_ref[...], preferred_element_type=_f32) + b_ref[...]
    o = jnp.where(o > 0, o, 0.01 * o)
    out_ref[...] = o
    i = pl.program_id(0)
    rows = i * BR + lax.broadcasted_iota(jnp.int32, (BR, 1), 0)
    hs_ref[...] = jnp.where(rows < N, o * ns_ref[...], 0.0)


def _tc_layer(part, nd, w, b, ns):
    return pl.pallas_call(
        _layer_body,
        grid=(NP // BR,),
        in_specs=[
            pl.BlockSpec((NC, BR, D), lambda i: (0, i, 0)),
            pl.BlockSpec((BR, 1), lambda i: (i, 0)),
            pl.BlockSpec((D, D), lambda i: (0, 0)),
            pl.BlockSpec((1, D), lambda i: (0, 0)),
            pl.BlockSpec((BR, 1), lambda i: (i, 0)),
        ],
        out_specs=[
            pl.BlockSpec((BR, D), lambda i: (i, 0)),
            pl.BlockSpec((BR, D), lambda i: (i, 0)),
        ],
        out_shape=[
            jax.ShapeDtypeStruct((NP, D), _f32),
            jax.ShapeDtypeStruct((NP, D), _f32),
        ],
    )(part, nd, w, b, ns)


def kernel(x, edge_index, W0, b0, W1, b1):
    pad = N + (jnp.arange(EP - E, dtype=jnp.int32) % (NP - N))
    src = jnp.concatenate([edge_index[0], pad])
    dst = jnp.concatenate([edge_index[1], pad])

    odp, idp = _sc_deg(src, dst)
    odp = odp.reshape(NC, NP, 1)
    idp = idp.reshape(NC, NP, 1)

    h0s, ns, nd = _tc_scale(x, odp, idp)

    hs = h0s
    out = None
    for w, b in ((W0, b0), (W1, b1)):
        p = _sc_agg(hs, src, dst)
        out, hs = _tc_layer(p, nd, w, b.reshape(1, D), ns)
    return out[:N]


# speedup vs baseline: 1.0126x; 1.0126x over previous
import jax
import jax.numpy as jnp
from jax import lax
from jax.experimental import pallas as pl
from jax.experimental.pallas import tpu as pltpu
from jax.experimental.pallas import tpu_sc as plsc

N = 10000
E = 320000
EP = 327680
D = 128
NP = 10240
NC = 2
NS = 16
NW = NC * NS
EPW = EP // NW
CH = 128
NCHUNK = EPW // CH
RPT = NP // NS
BR = 640

_f32 = jnp.float32
_mesh = plsc.VectorSubcoreMesh(
    core_axis_name="c", subcore_axis_name="s", num_cores=NC, num_subcores=NS
)


def _deg_body(src_hbm, dst_hbm, od_out, id_out,
              src_v, dst_v, si_v, di_v, ones_v, z16_v, cb_v, sp_od, sp_id):
    c = lax.axis_index("c")
    s = lax.axis_index("s")
    wid = s * NC + c

    z16_v[...] = jnp.zeros((16,), _f32)
    for i in range(CH // 16):
        ones_v[pl.ds(16 * i, 16)] = jnp.ones((16,), _f32)

    def _z(k, _):
        pltpu.sync_copy(z16_v, sp_od.at[pl.ds(s * RPT + 16 * k, 16)])
        pltpu.sync_copy(z16_v, sp_id.at[pl.ds(s * RPT + 16 * k, 16)])
        return _
    lax.fori_loop(0, RPT // 16, _z, None)

    pltpu.sync_copy(src_hbm.at[pl.ds(wid * EPW, EPW)], src_v)
    pltpu.sync_copy(dst_hbm.at[pl.ds(wid * EPW, EPW)], dst_v)
    plsc.subcore_barrier()

    def _acc(j, _):
        for k in range(CH // 16):
            si_v[pl.ds(16 * k, 16)] = src_v[pl.ds(j * CH + 16 * k, 16)]
            di_v[pl.ds(16 * k, 16)] = dst_v[pl.ds(j * CH + 16 * k, 16)]
        pltpu.sync_copy(ones_v, sp_od.at[si_v], add=True)
        pltpu.sync_copy(ones_v, sp_id.at[di_v], add=True)
        return _
    lax.fori_loop(0, NCHUNK, _acc, None)
    plsc.subcore_barrier()

    pltpu.sync_copy(sp_od.at[pl.ds(s * RPT, RPT)], cb_v)
    pltpu.sync_copy(cb_v, od_out.at[c, pl.ds(s * RPT, RPT)])
    pltpu.sync_copy(sp_id.at[pl.ds(s * RPT, RPT)], cb_v)
    pltpu.sync_copy(cb_v, id_out.at[c, pl.ds(s * RPT, RPT)])


_sc_deg = pl.kernel(
    _deg_body,
    out_type=[jax.ShapeDtypeStruct((NC, NP), _f32),
              jax.ShapeDtypeStruct((NC, NP), _f32)],
    mesh=_mesh,
    scratch_types=[
        pltpu.VMEM((EPW,), jnp.int32),
        pltpu.VMEM((EPW,), jnp.int32),
        pltpu.VMEM((CH,), jnp.int32),
        pltpu.VMEM((CH,), jnp.int32),
        pltpu.VMEM((CH,), _f32),
        pltpu.VMEM((16,), _f32),
        pltpu.VMEM((RPT,), _f32),
        pltpu.VMEM_SHARED((NP,), _f32),
        pltpu.VMEM_SHARED((NP,), _f32),
    ],
)


def _agg_body(h_hbm, src_hbm, dst_hbm, part_out,
              si0, si1, di0, di1, rb0, rb1, zb_v,
              ssi0, ssi1, sdi0, sdi1, sg0, sg1, sp_agg):
    c = lax.axis_index("c")
    s = lax.axis_index("s")
    wid = s * NC + c
    base = wid * EPW

    for r in range(16):
        for q in range(D // 16):
            zb_v[r, pl.ds(16 * q, 16)] = jnp.zeros((16,), _f32)

    def _z(k, _):
        pltpu.sync_copy(zb_v, sp_agg.at[pl.ds(s * RPT + 16 * k, 16)])
        return _
    lax.fori_loop(0, RPT // 16, _z, None)
    plsc.subcore_barrier()

    def _idx(j, sv, dv, ss, sd):
        pltpu.async_copy(src_hbm.at[pl.ds(base + j * CH, CH)], sv, ss)
        pltpu.async_copy(dst_hbm.at[pl.ds(base + j * CH, CH)], dv, sd)

    def _wi(v, sem):
        pltpu.make_async_copy(src_hbm.at[pl.ds(0, CH)], v, sem).wait()

    def _wg(rb, sem):
        pltpu.make_async_copy(h_hbm.at[pl.ds(0, CH)], rb, sem).wait()

    _idx(0, si0, di0, ssi0, sdi0)
    _wi(si0, ssi0)
    pltpu.async_copy(h_hbm.at[si0], rb0, sg0)
    _idx(1, si1, di1, ssi1, sdi1)

    def _pipe(t, _):
        j0 = 2 * t
        _wi(si1, ssi1)
        pltpu.async_copy(h_hbm.at[si1], rb1, sg1)
        _wg(rb0, sg0)
        _wi(di0, sdi0)
        pltpu.sync_copy(rb0, sp_agg.at[di0], add=True)
        _idx(j0 + 2, si0, di0, ssi0, sdi0)
        _wg(rb1, sg1)
        _wi(di1, sdi1)
        pltpu.sync_copy(rb1, sp_agg.at[di1], add=True)
        _wi(si0, ssi0)
        pltpu.async_copy(h_hbm.at[si0], rb0, sg0)
        _idx(j0 + 3, si1, di1, ssi1, sdi1)
        return _
    lax.fori_loop(0, NCHUNK // 2 - 1, _pipe, None)

    _wi(si1, ssi1)
    pltpu.async_copy(h_hbm.at[si1], rb1, sg1)
    _wg(rb0, sg0)
    _wi(di0, sdi0)
    pltpu.sync_copy(rb0, sp_agg.at[di0], add=True)
    _wg(rb1, sg1)
    _wi(di1, sdi1)
    pltpu.sync_copy(rb1, sp_agg.at[di1], add=True)
    plsc.subcore_barrier()

    def _out(k, _):
        b2 = s * RPT + CH * k
        pltpu.sync_copy(sp_agg.at[pl.ds(b2, CH)], rb0)
        pltpu.sync_copy(rb0, part_out.at[c, pl.ds(b2, CH)])
        return _
    lax.fori_loop(0, RPT // CH, _out, None)


_sc_agg = pl.kernel(
    _agg_body,
    out_type=jax.ShapeDtypeStruct((NC, NP, D), _f32),
    mesh=_mesh,
    scratch_types=[
        pltpu.VMEM((CH,), jnp.int32),
        pltpu.VMEM((CH,), jnp.int32),
        pltpu.VMEM((CH,), jnp.int32),
        pltpu.VMEM((CH,), jnp.int32),
        pltpu.VMEM((CH, D), _f32),
        pltpu.VMEM((CH, D), _f32),
        pltpu.VMEM((16, D), _f32),
        pltpu.SemaphoreType.DMA,
        pltpu.SemaphoreType.DMA,
        pltpu.SemaphoreType.DMA,
        pltpu.SemaphoreType.DMA,
        pltpu.SemaphoreType.DMA,
        pltpu.SemaphoreType.DMA,
        pltpu.VMEM_SHARED((NP, D), _f32),
    ],
)


def _scale_body(x_ref, odp_ref, idp_ref, hs_ref, ns_ref, nd_ref):
    od = odp_ref[0] + odp_ref[1]
    ind = idp_ref[0] + idp_ref[1]
    ns = lax.rsqrt(jnp.maximum(od, 1.0))
    nd = lax.rsqrt(jnp.maximum(ind, 1.0))
    i = pl.program_id(0)
    rows = i * BR + lax.broadcasted_iota(jnp.int32, (BR, 1), 0)
    hs_ref[...] = jnp.where(rows < N, x_ref[...] * ns, 0.0)
    ns_ref[...] = ns
    nd_ref[...] = nd


def _tc_scale(x, odp, idp):
    return pl.pallas_call(
        _scale_body,
        grid=(NP // BR,),
        in_specs=[
            pl.BlockSpec((BR, D), lambda i: (i, 0)),
            pl.BlockSpec((NC, BR, 1), lambda i: (0, i, 0)),
            pl.BlockSpec((NC, BR, 1), lambda i: (0, i, 0)),
        ],
        out_specs=[
            pl.BlockSpec((BR, D), lambda i: (i, 0)),
            pl.BlockSpec((BR, 1), lambda i: (i, 0)),
            pl.BlockSpec((BR, 1), lambda i: (i, 0)),
        ],
        out_shape=[
            jax.ShapeDtypeStruct((NP, D), _f32),
            jax.ShapeDtypeStruct((NP, 1), _f32),
            jax.ShapeDtypeStruct((NP, 1), _f32),
        ],
    )(x, odp, idp)


def _layer_body(part_ref, nd_ref, w_ref, b_ref, ns_ref, out_ref, hs_ref):
    agg = (part_ref[0] + part_ref[1]) * nd_ref[...]
    o = jnp.dot(agg, w_ref[...], preferred_element_type=_f32) + b_ref[...]
    o = jnp.where(o > 0, o, 0.01 * o)
    out_ref[...] = o
    i = pl.program_id(0)
    rows = i * BR + lax.broadcasted_iota(jnp.int32, (BR, 1), 0)
    hs_ref[...] = jnp.where(rows < N, o * ns_ref[...], 0.0)


def _tc_layer(part, nd, w, b, ns):
    return pl.pallas_call(
        _layer_body,
        grid=(NP // BR,),
        in_specs=[
            pl.BlockSpec((NC, BR, D), lambda i: (0, i, 0)),
            pl.BlockSpec((BR, 1), lambda i: (i, 0)),
            pl.BlockSpec((D, D), lambda i: (0, 0)),
            pl.BlockSpec((1, D), lambda i: (0, 0)),
            pl.BlockSpec((BR, 1), lambda i: (i, 0)),
        ],
        out_specs=[
            pl.BlockSpec((BR, D), lambda i: (i, 0)),
            pl.BlockSpec((BR, D), lambda i: (i, 0)),
        ],
        out_shape=[
            jax.ShapeDtypeStruct((NP, D), _f32),
            jax.ShapeDtypeStruct((NP, D), _f32),
        ],
    )(part, nd, w, b, ns)


def kernel(x, edge_index, W0, b0, W1, b1):
    pad = N + (jnp.arange(EP - E, dtype=jnp.int32) % (NP - N))
    src = jnp.concatenate([edge_index[0], pad])
    dst = jnp.concatenate([edge_index[1], pad])

    odp, idp = _sc_deg(src, dst)
    odp = odp.reshape(NC, NP, 1)
    idp = idp.reshape(NC, NP, 1)

    h0s, ns, nd = _tc_scale(x, odp, idp)

    hs = h0s
    out = None
    for w, b in ((W0, b0), (W1, b1)):
        p = _sc_agg(hs, src, dst)
        out, hs = _tc_layer(p, nd, w, b.reshape(1, D), ns)
    return out[:N]


# speedup vs baseline: 1.0134x; 1.0008x over previous
import jax
import jax.numpy as jnp
from jax import lax
from jax.experimental import pallas as pl
from jax.experimental.pallas import tpu as pltpu
from jax.experimental.pallas import tpu_sc as plsc

N = 10000
E = 320000
EP = 327680
D = 128
NP = 10240
NC = 2
NS = 16
NW = NC * NS
EPW = EP // NW
CH = 128
NCHUNK = EPW // CH
RPT = NP // NS
BR = 640

_f32 = jnp.float32
_mesh = plsc.VectorSubcoreMesh(
    core_axis_name="c", subcore_axis_name="s", num_cores=NC, num_subcores=NS
)


def _deg_body(src_hbm, dst_hbm, od_out, id_out,
              src_v, dst_v, si_v, di_v, ones_v, z16_v, cb_v, sp_od, sp_id):
    c = lax.axis_index("c")
    s = lax.axis_index("s")
    wid = s * NC + c

    z16_v[...] = jnp.zeros((16,), _f32)
    for i in range(CH // 16):
        ones_v[pl.ds(16 * i, 16)] = jnp.ones((16,), _f32)

    def _z(k, _):
        pltpu.sync_copy(z16_v, sp_od.at[pl.ds(s * RPT + 16 * k, 16)])
        pltpu.sync_copy(z16_v, sp_id.at[pl.ds(s * RPT + 16 * k, 16)])
        return _
    lax.fori_loop(0, RPT // 16, _z, None)

    pltpu.sync_copy(src_hbm.at[pl.ds(wid * EPW, EPW)], src_v)
    pltpu.sync_copy(dst_hbm.at[pl.ds(wid * EPW, EPW)], dst_v)
    plsc.subcore_barrier()

    def _acc(j, _):
        for k in range(CH // 16):
            si_v[pl.ds(16 * k, 16)] = src_v[pl.ds(j * CH + 16 * k, 16)]
            di_v[pl.ds(16 * k, 16)] = dst_v[pl.ds(j * CH + 16 * k, 16)]
        pltpu.sync_copy(ones_v, sp_od.at[si_v], add=True)
        pltpu.sync_copy(ones_v, sp_id.at[di_v], add=True)
        return _
    lax.fori_loop(0, NCHUNK, _acc, None)
    plsc.subcore_barrier()

    pltpu.sync_copy(sp_od.at[pl.ds(s * RPT, RPT)], cb_v)
    pltpu.sync_copy(cb_v, od_out.at[c, pl.ds(s * RPT, RPT)])
    pltpu.sync_copy(sp_id.at[pl.ds(s * RPT, RPT)], cb_v)
    pltpu.sync_copy(cb_v, id_out.at[c, pl.ds(s * RPT, RPT)])


_sc_deg = pl.kernel(
    _deg_body,
    out_type=[jax.ShapeDtypeStruct((NC, NP), _f32),
              jax.ShapeDtypeStruct((NC, NP), _f32)],
    mesh=_mesh,
    scratch_types=[
        pltpu.VMEM((EPW,), jnp.int32),
        pltpu.VMEM((EPW,), jnp.int32),
        pltpu.VMEM((CH,), jnp.int32),
        pltpu.VMEM((CH,), jnp.int32),
        pltpu.VMEM((CH,), _f32),
        pltpu.VMEM((16,), _f32),
        pltpu.VMEM((RPT,), _f32),
        pltpu.VMEM_SHARED((NP,), _f32),
        pltpu.VMEM_SHARED((NP,), _f32),
    ],
)


def _agg_body(h_hbm, src_hbm, dst_hbm, part_out,
              si0, si1, di0, di1, rb0, rb1, zb_v,
              ssi0, ssi1, sdi0, sdi1, sg0, sg1, sp_agg):
    c = lax.axis_index("c")
    s = lax.axis_index("s")
    wid = s * NC + c
    base = wid * EPW

    for r in range(16):
        for q in range(D // 16):
            zb_v[r, pl.ds(16 * q, 16)] = jnp.zeros((16,), _f32)

    def _z(k, _):
        pltpu.sync_copy(zb_v, sp_agg.at[pl.ds(s * RPT + 16 * k, 16)])
        return _
    lax.fori_loop(0, RPT // 16, _z, None)
    plsc.subcore_barrier()

    def _idx(j, sv, dv, ss, sd):
        pltpu.async_copy(src_hbm.at[pl.ds(base + j * CH, CH)], sv, ss)
        pltpu.async_copy(dst_hbm.at[pl.ds(base + j * CH, CH)], dv, sd)

    def _wi(v, sem):
        pltpu.make_async_copy(src_hbm.at[pl.ds(0, CH)], v, sem).wait()

    def _wg(rb, sem):
        pltpu.make_async_copy(h_hbm.at[pl.ds(0, CH)], rb, sem).wait()

    _idx(0, si0, di0, ssi0, sdi0)
    _wi(si0, ssi0)
    pltpu.async_copy(h_hbm.at[si0], rb0, sg0)
    _idx(1, si1, di1, ssi1, sdi1)

    def _pipe(t, _):
        j0 = 2 * t
        _wi(si1, ssi1)
        pltpu.async_copy(h_hbm.at[si1], rb1, sg1)
        _wg(rb0, sg0)
        _wi(di0, sdi0)
        pltpu.sync_copy(rb0, sp_agg.at[di0], add=True)
        _idx(j0 + 2, si0, di0, ssi0, sdi0)
        _wg(rb1, sg1)
        _wi(di1, sdi1)
        pltpu.sync_copy(rb1, sp_agg.at[di1], add=True)
        _wi(si0, ssi0)
        pltpu.async_copy(h_hbm.at[si0], rb0, sg0)
        _idx(j0 + 3, si1, di1, ssi1, sdi1)
        return _
    lax.fori_loop(0, NCHUNK // 2 - 1, _pipe, None)

    _wi(si1, ssi1)
    pltpu.async_copy(h_hbm.at[si1], rb1, sg1)
    _wg(rb0, sg0)
    _wi(di0, sdi0)
    pltpu.sync_copy(rb0, sp_agg.at[di0], add=True)
    _wg(rb1, sg1)
    _wi(di1, sdi1)
    pltpu.sync_copy(rb1, sp_agg.at[di1], add=True)
    plsc.subcore_barrier()

    pltpu.sync_copy(sp_agg.at[pl.ds(s * RPT, RPT)],
                    part_out.at[c, pl.ds(s * RPT, RPT)])


_sc_agg = pl.kernel(
    _agg_body,
    out_type=jax.ShapeDtypeStruct((NC, NP, D), _f32),
    mesh=_mesh,
    scratch_types=[
        pltpu.VMEM((CH,), jnp.int32),
        pltpu.VMEM((CH,), jnp.int32),
        pltpu.VMEM((CH,), jnp.int32),
        pltpu.VMEM((CH,), jnp.int32),
        pltpu.VMEM((CH, D), _f32),
        pltpu.VMEM((CH, D), _f32),
        pltpu.VMEM((16, D), _f32),
        pltpu.SemaphoreType.DMA,
        pltpu.SemaphoreType.DMA,
        pltpu.SemaphoreType.DMA,
        pltpu.SemaphoreType.DMA,
        pltpu.SemaphoreType.DMA,
        pltpu.SemaphoreType.DMA,
        pltpu.VMEM_SHARED((NP, D), _f32),
    ],
)


def _scale_body(x_ref, odp_ref, idp_ref, hs_ref, ns_ref, nd_ref):
    od = odp_ref[0] + odp_ref[1]
    ind = idp_ref[0] + idp_ref[1]
    ns = lax.rsqrt(jnp.maximum(od, 1.0))
    nd = lax.rsqrt(jnp.maximum(ind, 1.0))
    i = pl.program_id(0)
    rows = i * BR + lax.broadcasted_iota(jnp.int32, (BR, 1), 0)
    hs_ref[...] = jnp.where(rows < N, x_ref[...] * ns, 0.0)
    ns_ref[...] = ns
    nd_ref[...] = nd


def _tc_scale(x, odp, idp):
    return pl.pallas_call(
        _scale_body,
        grid=(NP // BR,),
        in_specs=[
            pl.BlockSpec((BR, D), lambda i: (i, 0)),
            pl.BlockSpec((NC, BR, 1), lambda i: (0, i, 0)),
            pl.BlockSpec((NC, BR, 1), lambda i: (0, i, 0)),
        ],
        out_specs=[
            pl.BlockSpec((BR, D), lambda i: (i, 0)),
            pl.BlockSpec((BR, 1), lambda i: (i, 0)),
            pl.BlockSpec((BR, 1), lambda i: (i, 0)),
        ],
        out_shape=[
            jax.ShapeDtypeStruct((NP, D), _f32),
            jax.ShapeDtypeStruct((NP, 1), _f32),
            jax.ShapeDtypeStruct((NP, 1), _f32),
        ],
    )(x, odp, idp)


def _layer_body(part_ref, nd_ref, w_ref, b_ref, ns_ref, out_ref, hs_ref):
    agg = (part_ref[0] + part_ref[1]) * nd_ref[...]
    o = jnp.dot(agg, w_ref[...], preferred_element_type=_f32) + b_ref[...]
    o = jnp.where(o > 0, o, 0.01 * o)
    out_ref[...] = o
    i = pl.program_id(0)
    rows = i * BR + lax.broadcasted_iota(jnp.int32, (BR, 1), 0)
    hs_ref[...] = jnp.where(rows < N, o * ns_ref[...], 0.0)


def _tc_layer(part, nd, w, b, ns):
    return pl.pallas_call(
        _layer_body,
        grid=(NP // BR,),
        in_specs=[
            pl.BlockSpec((NC, BR, D), lambda i: (0, i, 0)),
            pl.BlockSpec((BR, 1), lambda i: (i, 0)),
            pl.BlockSpec((D, D), lambda i: (0, 0)),
            pl.BlockSpec((1, D), lambda i: (0, 0)),
            pl.BlockSpec((BR, 1), lambda i: (i, 0)),
        ],
        out_specs=[
            pl.BlockSpec((BR, D), lambda i: (i, 0)),
            pl.BlockSpec((BR, D), lambda i: (i, 0)),
        ],
        out_shape=[
            jax.ShapeDtypeStruct((NP, D), _f32),
            jax.ShapeDtypeStruct((NP, D), _f32),
        ],
    )(part, nd, w, b, ns)


def kernel(x, edge_index, W0, b0, W1, b1):
    pad = N + (jnp.arange(EP - E, dtype=jnp.int32) % (NP - N))
    src = jnp.concatenate([edge_index[0], pad])
    dst = jnp.concatenate([edge_index[1], pad])

    odp, idp = _sc_deg(src, dst)
    odp = odp.reshape(NC, NP, 1)
    idp = idp.reshape(NC, NP, 1)

    h0s, ns, nd = _tc_scale(x, odp, idp)

    hs = h0s
    out = None
    for w, b in ((W0, b0), (W1, b1)):
        p = _sc_agg(hs, src, dst)
        out, hs = _tc_layer(p, nd, w, b.reshape(1, D), ns)
    return out[:N]


# speedup vs baseline: 1.0222x; 1.0086x over previous
import jax
import jax.numpy as jnp
from jax import lax
from jax.experimental import pallas as pl
from jax.experimental.pallas import tpu as pltpu
from jax.experimental.pallas import tpu_sc as plsc

N = 10000
E = 320000
EP = 327680
D = 128
NP = 10240
NC = 2
NS = 16
NW = NC * NS
EPW = EP // NW
CH = 128
NCHUNK = EPW // CH
RPT = NP // NS
BR = 640

_f32 = jnp.float32
_mesh = plsc.VectorSubcoreMesh(
    core_axis_name="c", subcore_axis_name="s", num_cores=NC, num_subcores=NS
)


def _deg_body(src_hbm, dst_hbm, od_out, id_out,
              src_v, dst_v, si_v, di_v, ones_v, cb_v, sp_od, sp_id):
    c = lax.axis_index("c")
    s = lax.axis_index("s")
    wid = s * NC + c

    for i in range(CH // 16):
        ones_v[pl.ds(16 * i, 16)] = jnp.ones((16,), _f32)
    for i in range(RPT // 16):
        cb_v[pl.ds(16 * i, 16)] = jnp.zeros((16,), _f32)

    pltpu.sync_copy(cb_v, sp_od.at[pl.ds(s * RPT, RPT)])
    pltpu.sync_copy(cb_v, sp_id.at[pl.ds(s * RPT, RPT)])

    pltpu.sync_copy(src_hbm.at[pl.ds(wid * EPW, EPW)], src_v)
    pltpu.sync_copy(dst_hbm.at[pl.ds(wid * EPW, EPW)], dst_v)
    plsc.subcore_barrier()

    def _acc(j, _):
        for k in range(CH // 16):
            si_v[pl.ds(16 * k, 16)] = src_v[pl.ds(j * CH + 16 * k, 16)]
            di_v[pl.ds(16 * k, 16)] = dst_v[pl.ds(j * CH + 16 * k, 16)]
        pltpu.sync_copy(ones_v, sp_od.at[si_v], add=True)
        pltpu.sync_copy(ones_v, sp_id.at[di_v], add=True)
        return _
    lax.fori_loop(0, NCHUNK, _acc, None)
    plsc.subcore_barrier()

    pltpu.sync_copy(sp_od.at[pl.ds(s * RPT, RPT)], od_out.at[c, pl.ds(s * RPT, RPT)])
    pltpu.sync_copy(sp_id.at[pl.ds(s * RPT, RPT)], id_out.at[c, pl.ds(s * RPT, RPT)])


_sc_deg = pl.kernel(
    _deg_body,
    out_type=[jax.ShapeDtypeStruct((NC, NP), _f32),
              jax.ShapeDtypeStruct((NC, NP), _f32)],
    mesh=_mesh,
    scratch_types=[
        pltpu.VMEM((EPW,), jnp.int32),
        pltpu.VMEM((EPW,), jnp.int32),
        pltpu.VMEM((CH,), jnp.int32),
        pltpu.VMEM((CH,), jnp.int32),
        pltpu.VMEM((CH,), _f32),
        pltpu.VMEM((RPT,), _f32),
        pltpu.VMEM_SHARED((NP,), _f32),
        pltpu.VMEM_SHARED((NP,), _f32),
    ],
)


def _agg_body(h_hbm, src_hbm, dst_hbm, part_out,
              si0, si1, di0, di1, rb0, rb1,
              ssi0, ssi1, sdi0, sdi1, sg0, sg1, sp_agg):
    c = lax.axis_index("c")
    s = lax.axis_index("s")
    wid = s * NC + c
    base = wid * EPW

    for r in range(CH):
        for q in range(D // 16):
            rb1[r, pl.ds(16 * q, 16)] = jnp.zeros((16,), _f32)

    def _z(k, _):
        pltpu.sync_copy(rb1, sp_agg.at[pl.ds(s * RPT + CH * k, CH)])
        return _
    lax.fori_loop(0, RPT // CH, _z, None)
    plsc.subcore_barrier()

    def _idx(j, sv, dv, ss, sd):
        pltpu.async_copy(src_hbm.at[pl.ds(base + j * CH, CH)], sv, ss)
        pltpu.async_copy(dst_hbm.at[pl.ds(base + j * CH, CH)], dv, sd)

    def _wi(v, sem):
        pltpu.make_async_copy(src_hbm.at[pl.ds(0, CH)], v, sem).wait()

    def _wg(rb, sem):
        pltpu.make_async_copy(h_hbm.at[pl.ds(0, CH)], rb, sem).wait()

    _idx(0, si0, di0, ssi0, sdi0)
    _wi(si0, ssi0)
    pltpu.async_copy(h_hbm.at[si0], rb0, sg0)
    _idx(1, si1, di1, ssi1, sdi1)

    def _pipe(t, _):
        j0 = 2 * t
        _wi(si1, ssi1)
        pltpu.async_copy(h_hbm.at[si1], rb1, sg1)
        _wg(rb0, sg0)
        _wi(di0, sdi0)
        pltpu.sync_copy(rb0, sp_agg.at[di0], add=True)
        _idx(j0 + 2, si0, di0, ssi0, sdi0)
        _wg(rb1, sg1)
        _wi(di1, sdi1)
        pltpu.sync_copy(rb1, sp_agg.at[di1], add=True)
        _wi(si0, ssi0)
        pltpu.async_copy(h_hbm.at[si0], rb0, sg0)
        _idx(j0 + 3, si1, di1, ssi1, sdi1)
        return _
    lax.fori_loop(0, NCHUNK // 2 - 1, _pipe, None)

    _wi(si1, ssi1)
    pltpu.async_copy(h_hbm.at[si1], rb1, sg1)
    _wg(rb0, sg0)
    _wi(di0, sdi0)
    pltpu.sync_copy(rb0, sp_agg.at[di0], add=True)
    _wg(rb1, sg1)
    _wi(di1, sdi1)
    pltpu.sync_copy(rb1, sp_agg.at[di1], add=True)
    plsc.subcore_barrier()

    pltpu.sync_copy(sp_agg.at[pl.ds(s * RPT, RPT)],
                    part_out.at[c, pl.ds(s * RPT, RPT)])


_sc_agg = pl.kernel(
    _agg_body,
    out_type=jax.ShapeDtypeStruct((NC, NP, D), _f32),
    mesh=_mesh,
    scratch_types=[
        pltpu.VMEM((CH,), jnp.int32),
        pltpu.VMEM((CH,), jnp.int32),
        pltpu.VMEM((CH,), jnp.int32),
        pltpu.VMEM((CH,), jnp.int32),
        pltpu.VMEM((CH, D), _f32),
        pltpu.VMEM((CH, D), _f32),
        pltpu.SemaphoreType.DMA,
        pltpu.SemaphoreType.DMA,
        pltpu.SemaphoreType.DMA,
        pltpu.SemaphoreType.DMA,
        pltpu.SemaphoreType.DMA,
        pltpu.SemaphoreType.DMA,
        pltpu.VMEM_SHARED((NP, D), _f32),
    ],
)


def _scale_body(x_ref, odp_ref, idp_ref, hs_ref, ns_ref, nd_ref):
    od = odp_ref[0] + odp_ref[1]
    ind = idp_ref[0] + idp_ref[1]
    ns = lax.rsqrt(jnp.maximum(od, 1.0))
    nd = lax.rsqrt(jnp.maximum(ind, 1.0))
    i = pl.program_id(0)
    rows = i * BR + lax.broadcasted_iota(jnp.int32, (BR, 1), 0)
    hs_ref[...] = jnp.where(rows < N, x_ref[...] * ns, 0.0)
    ns_ref[...] = ns
    nd_ref[...] = nd


def _tc_scale(x, odp, idp):
    return pl.pallas_call(
        _scale_body,
        grid=(NP // BR,),
        in_specs=[
            pl.BlockSpec((BR, D), lambda i: (i, 0)),
            pl.BlockSpec((NC, BR, 1), lambda i: (0, i, 0)),
            pl.BlockSpec((NC, BR, 1), lambda i: (0, i, 0)),
        ],
        out_specs=[
            pl.BlockSpec((BR, D), lambda i: (i, 0)),
            pl.BlockSpec((BR, 1), lambda i: (i, 0)),
            pl.BlockSpec((BR, 1), lambda i: (i, 0)),
        ],
        out_shape=[
            jax.ShapeDtypeStruct((NP, D), _f32),
            jax.ShapeDtypeStruct((NP, 1), _f32),
            jax.ShapeDtypeStruct((NP, 1), _f32),
        ],
    )(x, odp, idp)


def _layer_body(part_ref, nd_ref, w_ref, b_ref, ns_ref, out_ref, hs_ref):
    agg = (part_ref[0] + part_ref[1]) * nd_ref[...]
    o = jnp.dot(agg, w_ref[...], preferred_element_type=_f32) + b_ref[...]
    o = jnp.where(o > 0, o, 0.01 * o)
    out_ref[...] = o
    i = pl.program_id(0)
    rows = i * BR + lax.broadcasted_iota(jnp.int32, (BR, 1), 0)
    hs_ref[...] = jnp.where(rows < N, o * ns_ref[...], 0.0)


def _tc_layer(part, nd, w, b, ns):
    return pl.pallas_call(
        _layer_body,
        grid=(NP // BR,),
        in_specs=[
            pl.BlockSpec((NC, BR, D), lambda i: (0, i, 0)),
            pl.BlockSpec((BR, 1), lambda i: (i, 0)),
            pl.BlockSpec((D, D), lambda i: (0, 0)),
            pl.BlockSpec((1, D), lambda i: (0, 0)),
            pl.BlockSpec((BR, 1), lambda i: (i, 0)),
        ],
        out_specs=[
            pl.BlockSpec((BR, D), lambda i: (i, 0)),
            pl.BlockSpec((BR, D), lambda i: (i, 0)),
        ],
        out_shape=[
            jax.ShapeDtypeStruct((NP, D), _f32),
            jax.ShapeDtypeStruct((NP, D), _f32),
        ],
    )(part, nd, w, b, ns)


def kernel(x, edge_index, W0, b0, W1, b1):
    pad = N + (jnp.arange(EP - E, dtype=jnp.int32) % (NP - N))
    src = jnp.concatenate([edge_index[0], pad])
    dst = jnp.concatenate([edge_index[1], pad])

    odp, idp = _sc_deg(src, dst)
    odp = odp.reshape(NC, NP, 1)
    idp = idp.reshape(NC, NP, 1)

    h0s, ns, nd = _tc_scale(x, odp, idp)

    hs = h0s
    out = None
    for w, b in ((W0, b0), (W1, b1)):
        p = _sc_agg(hs, src, dst)
        out, hs = _tc_layer(p, nd, w, b.reshape(1, D), ns)
    return out[:N]
